# baseline (speedup 1.0000x reference)
import functools

import jax
import jax.numpy as jnp
from jax import lax
from jax.experimental import pallas as pl
from jax.experimental.pallas import tpu as pltpu
from jax.experimental.pallas import tpu_sc as plsc

N = 10000
E = 320000
NG = 64
NP = 10240
HALF = NP // 2
R = 1280
NBLK = NP // R
W = 64
NSUB = 16
NCORE = 2
NW = NCORE * NSUB
CH = 164
EP = NW * CH * W
RPS = NP // NSUB
HR = HALF // NSUB
CH2 = EP // NSUB // W
CAP = (CH2 + 4) * W
PADSRC = N

_HI = lax.Precision.HIGHEST


def _dinv_of(deg_blk):
  deg = deg_blk[0, :, 0:1] + deg_blk[1, :, 0:1]
  return jnp.where(deg > 0, lax.rsqrt(deg), 0.0)


@functools.cache
def _make_compact():
  out_types = (
      jax.ShapeDtypeStruct((NCORE, NSUB, CAP), jnp.int32),
      jax.ShapeDtypeStruct((NCORE, NSUB, CAP), jnp.int32),
      jax.ShapeDtypeStruct((NCORE, NSUB, 16), jnp.int32),
  )

  @functools.partial(
      pl.kernel,
      out_type=out_types,
      mesh=plsc.VectorSubcoreMesh(core_axis_name="c", subcore_axis_name="s"),
      compiler_params=pltpu.CompilerParams(needs_layout_passes=False),
      scratch_types=[
          pltpu.VMEM((CH2 * W,), jnp.int32),
          pltpu.VMEM((CH2 * W,), jnp.int32),
          pltpu.VMEM((CAP,), jnp.int32),
          pltpu.VMEM((CAP,), jnp.int32),
          pltpu.VMEM((16,), jnp.int32),
      ],
  )
  def compact(src_hbm, dst_hbm, csrc_hbm, cdst_hbm, cnt_hbm,
              sbig, dbig, osrc, odst, cntv):
    cid = lax.axis_index("c")
    sid = lax.axis_index("s")
    lo = cid * HALF
    pltpu.sync_copy(src_hbm.at[sid], sbig)
    pltpu.sync_copy(dst_hbm.at[sid], dbig)

    @pl.loop(0, CAP, step=16)
    def _(i):
      osrc[pl.ds(i, 16)] = jnp.full((16,), PADSRC, jnp.int32)
      odst[pl.ds(i, 16)] = jnp.zeros((16,), jnp.int32)

    def body(i, off):
      sv = sbig[pl.ds(i, 16)]
      dv = dbig[pl.ds(i, 16)] - lo
      m = (dv >= 0) & (dv < HALF)
      plsc.store_compressed(osrc.at[pl.ds(off, 16)], sv, mask=m)
      plsc.store_compressed(odst.at[pl.ds(off, 16)], dv, mask=m)
      return off + jnp.sum(m.astype(jnp.int32))

    off = pl.loop(0, CH2 * W, step=16, init_carry=0)(body)

    nch = ((off + 4 * W - 1) // (4 * W)) * 4
    cntv[...] = jnp.full((16,), 1, jnp.int32) * nch
    pltpu.sync_copy(osrc, csrc_hbm.at[cid, sid])
    pltpu.sync_copy(odst, cdst_hbm.at[cid, sid])
    pltpu.sync_copy(cntv, cnt_hbm.at[cid, sid])

  return compact


@functools.cache
def _make_deg():
  @functools.partial(
      pl.kernel,
      out_type=jax.ShapeDtypeStruct((NCORE, NP, 128), jnp.float32),
      mesh=plsc.VectorSubcoreMesh(core_axis_name="c", subcore_axis_name="s"),
      scratch_types=[
          pltpu.VMEM_SHARED((NP, 128), jnp.float32),
          pltpu.VMEM((W,), jnp.int32),
          pltpu.VMEM((W,), jnp.int32),
          pltpu.VMEM((W, 128), jnp.float32),
          pltpu.SemaphoreType.DMA,
          pltpu.SemaphoreType.DMA,
      ],
  )
  def deg_kernel(dst_hbm, ones_hbm, zeros_hbm, out_hbm, acc, da, db, ones_v,
                 sem_a, sem_b):
    cid = lax.axis_index("c")
    sid = lax.axis_index("s")
    w = cid * NSUB + sid

    def idx_wait(dbuf, sem):
      pltpu.make_async_copy(dst_hbm.at[w, 0], dbuf, sem).wait()

    pltpu.sync_copy(ones_hbm, ones_v)
    pltpu.sync_copy(zeros_hbm, acc.at[pl.ds(sid * RPS, RPS)])
    plsc.subcore_barrier()

    pltpu.async_copy(dst_hbm.at[w, 0], da, sem_a)
    pltpu.async_copy(dst_hbm.at[w, 1], db, sem_b)

    @pl.loop(0, CH, step=2)
    def _(i):
      idx_wait(da, sem_a)
      pltpu.sync_copy(ones_v, acc.at[da], add=True)

      @pl.when(i + 2 < CH)
      def _():
        pltpu.async_copy(dst_hbm.at[w, i + 2], da, sem_a)

      idx_wait(db, sem_b)
      pltpu.sync_copy(ones_v, acc.at[db], add=True)

      @pl.when(i + 3 < CH)
      def _():
        pltpu.async_copy(dst_hbm.at[w, i + 3], db, sem_b)

    plsc.subcore_barrier()
    pltpu.sync_copy(acc.at[pl.ds(sid * RPS, RPS)],
                    out_hbm.at[cid, pl.ds(sid * RPS, RPS)])

  return deg_kernel


@functools.cache
def _make_agg(ncols, P, K):
  wide = ncols == 256
  arows = 2 * HALF if wide else HALF
  orows = 2 * NP if wide else NP
  hr = 2 * HR if wide else HR

  @functools.partial(
      pl.kernel,
      out_type=jax.ShapeDtypeStruct((P, orows, 128), jnp.float32),
      mesh=plsc.VectorSubcoreMesh(core_axis_name="c", subcore_axis_name="s"),
      compiler_params=pltpu.CompilerParams(needs_layout_passes=False),
      scratch_types=(
          [pltpu.VMEM_SHARED((arows, 128), jnp.float32)]
          + [pltpu.VMEM((W,), jnp.int32) for _ in range(K)]
          + [pltpu.VMEM((W,), jnp.int32) for _ in range(K)]
          + [pltpu.VMEM((2 * W,), jnp.int32) for _ in range(K)]
          + [pltpu.VMEM((W, ncols // 128, 128), jnp.float32)
             for _ in range(K)]
          + [pltpu.VMEM((16,), jnp.int32)]
          + [pltpu.SemaphoreType.DMA for _ in range(2 * K)]
      ),
  )
  def agg(g_hbm, csrc_hbm, cdst_hbm, cnt_hbm, zeros_hbm, out_hbm, acc, *scr):
    sbufs = scr[0:K]
    dbufs = scr[K:2 * K]
    ebufs = scr[2 * K:3 * K]
    rbufs = scr[3 * K:4 * K]
    cbuf = scr[4 * K]
    isems = scr[4 * K + 1:5 * K + 1]
    gsems = scr[5 * K + 1:6 * K + 1]
    cid = lax.axis_index("c")
    sid = lax.axis_index("s")

    pltpu.sync_copy(cnt_hbm.at[cid, sid], cbuf)
    nch = cbuf[...][0]

    def idx_load(i, b):
      pltpu.async_copy(csrc_hbm.at[cid, sid, pl.ds(i * W, W)],
                       sbufs[b], isems[b])
      pltpu.async_copy(cdst_hbm.at[cid, sid, pl.ds(i * W, W)],
                       dbufs[b], isems[b])

    def idx_wait(b):
      pltpu.make_async_copy(csrc_hbm.at[cid, sid, pl.ds(0, W)],
                            sbufs[b], isems[b]).wait()
      pltpu.make_async_copy(cdst_hbm.at[cid, sid, pl.ds(0, W)],
                            dbufs[b], isems[b]).wait()

    def gather_wait(b):
      pltpu.make_async_copy(g_hbm.at[sbufs[0]], rbufs[b], gsems[b]).wait()

    for p in range(P):
      pltpu.sync_copy(zeros_hbm, acc.at[pl.ds(sid * hr, hr)])

      def gather(b):
        iota = lax.broadcasted_iota(jnp.int32, (16,), 0)
        for j in range(W // 16):
          sl = pl.ds(j * 16, 16)
          if p:
            sbufs[b][sl] += p * NP
          if wide:
            d2 = dbufs[b][sl] * 2
            pos = iota * 2 + (j * 32)
            plsc.store_scatter(ebufs[b], [pos], d2)
            plsc.store_scatter(ebufs[b], [pos + 1], d2 + 1)
        pltpu.async_copy(g_hbm.at[sbufs[b]], rbufs[b], gsems[b])

      def scatter(b):
        if wide:
          pltpu.sync_copy(rbufs[b].reshape(2 * W, 128),
                          acc.at[ebufs[b]], add=True)
        else:
          pltpu.sync_copy(rbufs[b].reshape(W, 128),
                          acc.at[dbufs[b]], add=True)

      plsc.subcore_barrier()

      @pl.when(nch > 0)
      def _():
        for b in range(K):
          idx_load(b, b)
        for b in range(K - 1):
          idx_wait(b)
          gather(b)

        @pl.loop(0, nch, step=K)
        def _(i):
          for b in range(K):
            bg = (b + K - 1) % K

            @pl.when(i + b + K - 1 < nch)
            def _(b=b, bg=bg):
              idx_wait(bg)
              gather(bg)

            gather_wait(b)
            scatter(b)

            @pl.when(i + b + K < nch)
            def _(b=b):
              idx_load(i + b + K, b)

      plsc.subcore_barrier()
      pltpu.sync_copy(acc.at[pl.ds(sid * hr, hr)],
                      out_hbm.at[p, pl.ds(cid * (arows) + sid * hr, hr)])

  return agg


def _dense1_body(x_ref, w_ref, deg_ref, o_ref):
  dinv = _dinv_of(deg_ref[...])
  g = lax.dot_general(x_ref[...], w_ref[...], (((1,), (0,)), ((), ())),
                      precision=_HI, preferred_element_type=jnp.float32)
  o_ref[...] = g * dinv


def _dense1(xp, w1, deg2):
  return pl.pallas_call(
      _dense1_body,
      grid=(NBLK,),
      in_specs=[
          pl.BlockSpec((R, 128), lambda r: (r, 0)),
          pl.BlockSpec((128, 128), lambda r: (0, 0)),
          pl.BlockSpec((2, R, 128), lambda r: (0, r, 0)),
      ],
      out_specs=pl.BlockSpec((R, 128), lambda r: (r, 0)),
      out_shape=jax.ShapeDtypeStruct((NP, 128), jnp.float32),
  )(xp, w1, deg2)


def _mid_body(s_ref, deg_ref, b_ref, w_ref, o_ref):
  dinv = _dinv_of(deg_ref[...])
  a = jnp.maximum(s_ref[...] * dinv + b_ref[...], 0.0)
  g = lax.dot_general(a, w_ref[...], (((1,), (0,)), ((), ())),
                      precision=_HI, preferred_element_type=jnp.float32)
  g = g * dinv
  if len(o_ref.shape) == 2:
    o_ref[...] = g
  else:
    o_ref[0] = g


def _dense2(s1, deg2, b, w):
  return pl.pallas_call(
      _mid_body,
      grid=(NBLK, 2),
      in_specs=[
          pl.BlockSpec((R, 128), lambda r, c: (r, 0)),
          pl.BlockSpec((2, R, 128), lambda r, c: (0, r, 0)),
          pl.BlockSpec((1, 128), lambda r, c: (0, 0)),
          pl.BlockSpec((128, 128), lambda r, c: (0, c)),
      ],
      out_specs=pl.BlockSpec((R, 128), lambda r, c: (r, c)),
      out_shape=jax.ShapeDtypeStruct((NP, 256), jnp.float32),
  )(s1, deg2, b, w)


def _dense3(s2, deg2, b, w):
  return pl.pallas_call(
      _mid_body,
      grid=(NBLK, 4),
      in_specs=[
          pl.BlockSpec((R, 256), lambda r, c: (r, 0)),
          pl.BlockSpec((2, R, 128), lambda r, c: (0, r, 0)),
          pl.BlockSpec((1, 256), lambda r, c: (0, 0)),
          pl.BlockSpec((256, 128), lambda r, c: (0, c)),
      ],
      out_specs=pl.BlockSpec((1, R, 128), lambda r, c: (c // 2, r, c % 2)),
      out_shape=jax.ShapeDtypeStruct((2, NP, 256), jnp.float32),
  )(s2, deg2, b, w)


def _pool_body(acc_ref, deg_ref, b3_ref, bidx_ref, wp1_ref, bp1_ref,
               wp2_ref, bp2_ref, o_ref, pool_ref):
  r = pl.program_id(0)
  s_full = jnp.concatenate([acc_ref[0], acc_ref[1]], axis=1)
  dinv = _dinv_of(deg_ref[...])
  h = jnp.maximum(s_full * dinv + b3_ref[...], 0.0)
  hh = jnp.concatenate([h, jnp.ones((R, 128), jnp.float32)], axis=1)
  bi = bidx_ref[0, 0, :]
  oh_t = (bi[None, :] == lax.broadcasted_iota(jnp.int32, (NG, R), 0)
          ).astype(jnp.float32)
  contrib = lax.dot_general(oh_t, hh, (((1,), (0,)), ((), ())),
                            precision=_HI, preferred_element_type=jnp.float32)

  @pl.when(r == 0)
  def _():
    pool_ref[...] = contrib

  @pl.when(r > 0)
  def _():
    pool_ref[...] += contrib

  @pl.when(r == NBLK - 1)
  def _():
    pool = pool_ref[...]
    inv = 1.0 / jnp.maximum(pool[:, 512:640], 1.0)
    pooled = pool[:, 0:512] * jnp.concatenate([inv] * 4, axis=1)
    z = lax.dot_general(pooled, wp1_ref[...], (((1,), (0,)), ((), ())),
                        precision=_HI, preferred_element_type=jnp.float32)
    z = jnp.maximum(z + bp1_ref[...], 0.0)
    o = lax.dot_general(z, wp2_ref[...], (((1,), (0,)), ((), ())),
                        precision=_HI, preferred_element_type=jnp.float32)
    o_ref[...] = o + bp2_ref[...]


def _pool_mlp(acc3, deg2, b3, batch3, wp1, bp1, wp2, bp2):
  return pl.pallas_call(
      _pool_body,
      grid=(NBLK,),
      in_specs=[
          pl.BlockSpec((2, R, 256), lambda r: (0, r, 0)),
          pl.BlockSpec((2, R, 128), lambda r: (0, r, 0)),
          pl.BlockSpec((1, 512), lambda r: (0, 0)),
          pl.BlockSpec((1, 1, R), lambda r: (r, 0, 0)),
          pl.BlockSpec((512, 1024), lambda r: (0, 0)),
          pl.BlockSpec((1, 1024), lambda r: (0, 0)),
          pl.BlockSpec((1024, 128), lambda r: (0, 0)),
          pl.BlockSpec((1, 128), lambda r: (0, 0)),
      ],
      out_specs=pl.BlockSpec((NG, 128), lambda r: (0, 0)),
      out_shape=jax.ShapeDtypeStruct((NG, 128), jnp.float32),
      scratch_shapes=[pltpu.VMEM((NG, 640), jnp.float32)],
  )(acc3, deg2, b3, batch3, wp1, bp1, wp2, bp2)


def kernel(x, edge_index, batch_idx, W1, b1, W2, b2, W3, b3, Wp1, bp1,
           Wp2, bp2):
  loop = jnp.arange(N, dtype=jnp.int32)
  pad = jnp.full((EP - E - N,), NP - 1, dtype=jnp.int32)
  src = jnp.concatenate([edge_index[0], loop, pad])
  dst = jnp.concatenate([edge_index[1], loop, pad])
  dst3 = dst.reshape(NW, CH, W)
  srcF = src.reshape(NSUB, CH2 * W)
  dstF = dst.reshape(NSUB, CH2 * W)
  xp = jnp.pad(x, ((0, NP - N), (0, 0)))
  batch3 = jnp.pad(batch_idx, (0, NP - N),
                   constant_values=NG).reshape(NBLK, 1, R)
  ones128 = jnp.ones((W, 128), jnp.float32)
  zdeg = jnp.zeros((RPS, 128), jnp.float32)
  z128 = jnp.zeros((HR, 128), jnp.float32)
  z256 = jnp.zeros((2 * HR, 128), jnp.float32)

  csrc, cdst, cnt = _make_compact()(srcF, dstF)
  deg2 = _make_deg()(dst3, ones128, zdeg)

  g1 = _dense1(xp, W1, deg2)
  s1 = _make_agg(128, 1, 4)(g1.reshape(NP, 1, 128), csrc, cdst, cnt,
                            z128)

  g2 = _dense2(s1.reshape(NP, 128), deg2, b1.reshape(1, -1), W2)
  s2 = _make_agg(256, 1, 2)(g2.reshape(NP, 2, 128), csrc, cdst, cnt,
                            z256)

  g3 = _dense3(s2.reshape(NP, 256), deg2, b2.reshape(1, -1), W3)
  s3 = _make_agg(256, 2, 2)(g3.reshape(2 * NP, 2, 128), csrc, cdst, cnt,
                            z256)
  s3 = s3.reshape(2, NP, 256)

  return _pool_mlp(s3, deg2, b3.reshape(1, -1), batch3,
                   Wp1, bp1.reshape(1, -1), Wp2, bp2.reshape(1, -1))


# speedup vs baseline: 1.0456x; 1.0456x over previous
import functools

import jax
import jax.numpy as jnp
from jax import lax
from jax.experimental import pallas as pl
from jax.experimental.pallas import tpu as pltpu
from jax.experimental.pallas import tpu_sc as plsc

N = 10000
E = 320000
NG = 64
NP = 10240
HALF = NP // 2
R = 1280
NBLK = NP // R
W = 64
NSUB = 16
NCORE = 2
NW = NCORE * NSUB
CH = 164
EP = NW * CH * W
RPS = NP // NSUB
HR = HALF // NSUB
CH2 = EP // NSUB // W
CAP = (CH2 + 4) * W
PADSRC = N

_HI = lax.Precision.HIGHEST


def _dinv_of(deg_blk):
  deg = deg_blk[0, :, 0:1] + deg_blk[1, :, 0:1]
  return jnp.where(deg > 0, lax.rsqrt(deg), 0.0)


@functools.cache
def _make_compact():
  out_types = (
      jax.ShapeDtypeStruct((NCORE, NSUB, CAP), jnp.int32),
      jax.ShapeDtypeStruct((NCORE, NSUB, CAP), jnp.int32),
      jax.ShapeDtypeStruct((NCORE, NSUB, 16), jnp.int32),
  )

  @functools.partial(
      pl.kernel,
      out_type=out_types,
      mesh=plsc.VectorSubcoreMesh(core_axis_name="c", subcore_axis_name="s"),
      compiler_params=pltpu.CompilerParams(needs_layout_passes=False),
      scratch_types=[
          pltpu.VMEM((CH2 * W,), jnp.int32),
          pltpu.VMEM((CH2 * W,), jnp.int32),
          pltpu.VMEM((CAP,), jnp.int32),
          pltpu.VMEM((CAP,), jnp.int32),
          pltpu.VMEM((16,), jnp.int32),
      ],
  )
  def compact(src_hbm, dst_hbm, csrc_hbm, cdst_hbm, cnt_hbm,
              sbig, dbig, osrc, odst, cntv):
    cid = lax.axis_index("c")
    sid = lax.axis_index("s")
    lo = cid * HALF
    pltpu.sync_copy(src_hbm.at[sid], sbig)
    pltpu.sync_copy(dst_hbm.at[sid], dbig)

    @pl.loop(0, CAP, step=16)
    def _(i):
      osrc[pl.ds(i, 16)] = jnp.full((16,), PADSRC, jnp.int32)
      odst[pl.ds(i, 16)] = jnp.zeros((16,), jnp.int32)

    def body(i, off):
      sv = sbig[pl.ds(i, 16)]
      dv = dbig[pl.ds(i, 16)] - lo
      m = (dv >= 0) & (dv < HALF)
      plsc.store_compressed(osrc.at[pl.ds(off, 16)], sv, mask=m)
      plsc.store_compressed(odst.at[pl.ds(off, 16)], dv, mask=m)
      return off + jnp.sum(m.astype(jnp.int32))

    off = pl.loop(0, CH2 * W, step=16, init_carry=0)(body)

    nch = ((off + 4 * W - 1) // (4 * W)) * 4
    cntv[...] = jnp.full((16,), 1, jnp.int32) * nch
    pltpu.sync_copy(osrc, csrc_hbm.at[cid, sid])
    pltpu.sync_copy(odst, cdst_hbm.at[cid, sid])
    pltpu.sync_copy(cntv, cnt_hbm.at[cid, sid])

  return compact


@functools.cache
def _make_deg():
  @functools.partial(
      pl.kernel,
      out_type=jax.ShapeDtypeStruct((NCORE, NP, 128), jnp.float32),
      mesh=plsc.VectorSubcoreMesh(core_axis_name="c", subcore_axis_name="s"),
      scratch_types=[
          pltpu.VMEM_SHARED((NP, 128), jnp.float32),
          pltpu.VMEM((W,), jnp.int32),
          pltpu.VMEM((W,), jnp.int32),
          pltpu.VMEM((W, 128), jnp.float32),
          pltpu.SemaphoreType.DMA,
          pltpu.SemaphoreType.DMA,
      ],
  )
  def deg_kernel(dst_hbm, ones_hbm, zeros_hbm, out_hbm, acc, da, db, ones_v,
                 sem_a, sem_b):
    cid = lax.axis_index("c")
    sid = lax.axis_index("s")
    w = cid * NSUB + sid

    def idx_wait(dbuf, sem):
      pltpu.make_async_copy(dst_hbm.at[w, 0], dbuf, sem).wait()

    pltpu.sync_copy(ones_hbm, ones_v)
    pltpu.sync_copy(zeros_hbm, acc.at[pl.ds(sid * RPS, RPS)])
    plsc.subcore_barrier()

    pltpu.async_copy(dst_hbm.at[w, 0], da, sem_a)
    pltpu.async_copy(dst_hbm.at[w, 1], db, sem_b)

    @pl.loop(0, CH, step=2)
    def _(i):
      idx_wait(da, sem_a)
      pltpu.sync_copy(ones_v, acc.at[da], add=True)

      @pl.when(i + 2 < CH)
      def _():
        pltpu.async_copy(dst_hbm.at[w, i + 2], da, sem_a)

      idx_wait(db, sem_b)
      pltpu.sync_copy(ones_v, acc.at[db], add=True)

      @pl.when(i + 3 < CH)
      def _():
        pltpu.async_copy(dst_hbm.at[w, i + 3], db, sem_b)

    plsc.subcore_barrier()
    pltpu.sync_copy(acc.at[pl.ds(sid * RPS, RPS)],
                    out_hbm.at[cid, pl.ds(sid * RPS, RPS)])

  return deg_kernel


@functools.cache
def _make_agg(ncols, P, K):
  wide = ncols == 256
  arows = 2 * HALF if wide else HALF
  orows = 2 * NP if wide else NP
  hr = 2 * HR if wide else HR

  @functools.partial(
      pl.kernel,
      out_type=jax.ShapeDtypeStruct((P, orows, 128), jnp.float32),
      mesh=plsc.VectorSubcoreMesh(core_axis_name="c", subcore_axis_name="s"),
      compiler_params=pltpu.CompilerParams(needs_layout_passes=False),
      scratch_types=(
          [pltpu.VMEM_SHARED((arows, 128), jnp.float32)]
          + [pltpu.VMEM((W,), jnp.int32) for _ in range(K)]
          + [pltpu.VMEM((W,), jnp.int32) for _ in range(K)]
          + [pltpu.VMEM((2 * W if wide else W,), jnp.int32)
             for _ in range(K)]
          + [pltpu.VMEM((W, ncols // 128, 128), jnp.float32)
             for _ in range(K)]
          + [pltpu.VMEM((16,), jnp.int32)]
          + [pltpu.SemaphoreType.DMA for _ in range(3 * K)]
      ),
  )
  def agg(g_hbm, csrc_hbm, cdst_hbm, cnt_hbm, zeros_hbm, out_hbm, acc, *scr):
    sbufs = scr[0:K]
    dbufs = scr[K:2 * K]
    ebufs = scr[2 * K:3 * K]
    rbufs = scr[3 * K:4 * K]
    cbuf = scr[4 * K]
    isems = scr[4 * K + 1:5 * K + 1]
    gsems = scr[5 * K + 1:6 * K + 1]
    ssems = scr[6 * K + 1:7 * K + 1]
    cid = lax.axis_index("c")
    sid = lax.axis_index("s")

    pltpu.sync_copy(cnt_hbm.at[cid, sid], cbuf)
    nch = cbuf[...][0]

    def idx_load(i, b):
      pltpu.async_copy(csrc_hbm.at[cid, sid, pl.ds(i * W, W)],
                       sbufs[b], isems[b])
      pltpu.async_copy(cdst_hbm.at[cid, sid, pl.ds(i * W, W)],
                       dbufs[b], isems[b])

    def idx_wait(b):
      pltpu.make_async_copy(csrc_hbm.at[cid, sid, pl.ds(0, W)],
                            sbufs[b], isems[b]).wait()
      pltpu.make_async_copy(cdst_hbm.at[cid, sid, pl.ds(0, W)],
                            dbufs[b], isems[b]).wait()

    def gather_wait(b):
      pltpu.make_async_copy(g_hbm.at[sbufs[0]], rbufs[b], gsems[b]).wait()

    for p in range(P):
      pltpu.sync_copy(zeros_hbm, acc.at[pl.ds(sid * hr, hr)])

      def gather(b):
        iota = lax.broadcasted_iota(jnp.int32, (16,), 0)
        for j in range(W // 16):
          sl = pl.ds(j * 16, 16)
          if p:
            sbufs[b][sl] += p * NP
          if wide:
            d2 = dbufs[b][sl] * 2
            pos = iota * 2 + (j * 32)
            plsc.store_scatter(ebufs[b], [pos], d2)
            plsc.store_scatter(ebufs[b], [pos + 1], d2 + 1)
          else:
            ebufs[b][sl] = dbufs[b][sl]
        pltpu.async_copy(g_hbm.at[sbufs[b]], rbufs[b], gsems[b])

      def scatter(b):
        if wide:
          pltpu.async_copy(rbufs[b].reshape(2 * W, 128),
                          acc.at[ebufs[b]], ssems[b], add=True)
        else:
          pltpu.async_copy(rbufs[b].reshape(W, 128),
                          acc.at[ebufs[b]], ssems[b], add=True)

      def scatter_wait(b):
        if wide:
          pltpu.make_async_copy(rbufs[b].reshape(2 * W, 128),
                                acc.at[ebufs[b]], ssems[b]).wait()
        else:
          pltpu.make_async_copy(rbufs[b].reshape(W, 128),
                                acc.at[ebufs[b]], ssems[b]).wait()

      plsc.subcore_barrier()

      @pl.when(nch > 0)
      def _():
        for b in range(K):
          idx_load(b, b)
        for b in range(K - 1):
          idx_wait(b)
          gather(b)

        @pl.loop(0, nch, step=K)
        def _(i):
          for b in range(K):
            bg = (b + K - 1) % K

            @pl.when(i + b + K - 1 < nch)
            def _(b=b, bg=bg):
              @pl.when(i + b > 0)
              def _():
                scatter_wait(bg)

              idx_wait(bg)
              gather(bg)

            gather_wait(b)
            scatter(b)

            @pl.when(i + b + K < nch)
            def _(b=b):
              idx_load(i + b + K, b)

        for b in range(K):
          scatter_wait(b)

      plsc.subcore_barrier()
      pltpu.sync_copy(acc.at[pl.ds(sid * hr, hr)],
                      out_hbm.at[p, pl.ds(cid * (arows) + sid * hr, hr)])

  return agg


def _dense1_body(x_ref, w_ref, deg_ref, o_ref):
  dinv = _dinv_of(deg_ref[...])
  g = lax.dot_general(x_ref[...], w_ref[...], (((1,), (0,)), ((), ())),
                      precision=_HI, preferred_element_type=jnp.float32)
  o_ref[...] = g * dinv


def _dense1(xp, w1, deg2):
  return pl.pallas_call(
      _dense1_body,
      grid=(NBLK,),
      in_specs=[
          pl.BlockSpec((R, 128), lambda r: (r, 0)),
          pl.BlockSpec((128, 128), lambda r: (0, 0)),
          pl.BlockSpec((2, R, 128), lambda r: (0, r, 0)),
      ],
      out_specs=pl.BlockSpec((R, 128), lambda r: (r, 0)),
      out_shape=jax.ShapeDtypeStruct((NP, 128), jnp.float32),
  )(xp, w1, deg2)


def _mid_body(s_ref, deg_ref, b_ref, w_ref, o_ref):
  dinv = _dinv_of(deg_ref[...])
  a = jnp.maximum(s_ref[...] * dinv + b_ref[...], 0.0)
  g = lax.dot_general(a, w_ref[...], (((1,), (0,)), ((), ())),
                      precision=_HI, preferred_element_type=jnp.float32)
  g = g * dinv
  if len(o_ref.shape) == 2:
    o_ref[...] = g
  else:
    o_ref[0] = g


def _dense2(s1, deg2, b, w):
  return pl.pallas_call(
      _mid_body,
      grid=(NBLK, 2),
      in_specs=[
          pl.BlockSpec((R, 128), lambda r, c: (r, 0)),
          pl.BlockSpec((2, R, 128), lambda r, c: (0, r, 0)),
          pl.BlockSpec((1, 128), lambda r, c: (0, 0)),
          pl.BlockSpec((128, 128), lambda r, c: (0, c)),
      ],
      out_specs=pl.BlockSpec((R, 128), lambda r, c: (r, c)),
      out_shape=jax.ShapeDtypeStruct((NP, 256), jnp.float32),
  )(s1, deg2, b, w)


def _dense3(s2, deg2, b, w):
  return pl.pallas_call(
      _mid_body,
      grid=(NBLK, 4),
      in_specs=[
          pl.BlockSpec((R, 256), lambda r, c: (r, 0)),
          pl.BlockSpec((2, R, 128), lambda r, c: (0, r, 0)),
          pl.BlockSpec((1, 256), lambda r, c: (0, 0)),
          pl.BlockSpec((256, 128), lambda r, c: (0, c)),
      ],
      out_specs=pl.BlockSpec((1, R, 128), lambda r, c: (c // 2, r, c % 2)),
      out_shape=jax.ShapeDtypeStruct((2, NP, 256), jnp.float32),
  )(s2, deg2, b, w)


def _pool_body(acc_ref, deg_ref, b3_ref, bidx_ref, wp1_ref, bp1_ref,
               wp2_ref, bp2_ref, o_ref, pool_ref):
  r = pl.program_id(0)
  s_full = jnp.concatenate([acc_ref[0], acc_ref[1]], axis=1)
  dinv = _dinv_of(deg_ref[...])
  h = jnp.maximum(s_full * dinv + b3_ref[...], 0.0)
  hh = jnp.concatenate([h, jnp.ones((R, 128), jnp.float32)], axis=1)
  bi = bidx_ref[0, 0, :]
  oh_t = (bi[None, :] == lax.broadcasted_iota(jnp.int32, (NG, R), 0)
          ).astype(jnp.float32)
  contrib = lax.dot_general(oh_t, hh, (((1,), (0,)), ((), ())),
                            precision=_HI, preferred_element_type=jnp.float32)

  @pl.when(r == 0)
  def _():
    pool_ref[...] = contrib

  @pl.when(r > 0)
  def _():
    pool_ref[...] += contrib

  @pl.when(r == NBLK - 1)
  def _():
    pool = pool_ref[...]
    inv = 1.0 / jnp.maximum(pool[:, 512:640], 1.0)
    pooled = pool[:, 0:512] * jnp.concatenate([inv] * 4, axis=1)
    z = lax.dot_general(pooled, wp1_ref[...], (((1,), (0,)), ((), ())),
                        precision=_HI, preferred_element_type=jnp.float32)
    z = jnp.maximum(z + bp1_ref[...], 0.0)
    o = lax.dot_general(z, wp2_ref[...], (((1,), (0,)), ((), ())),
                        precision=_HI, preferred_element_type=jnp.float32)
    o_ref[...] = o + bp2_ref[...]


def _pool_mlp(acc3, deg2, b3, batch3, wp1, bp1, wp2, bp2):
  return pl.pallas_call(
      _pool_body,
      grid=(NBLK,),
      in_specs=[
          pl.BlockSpec((2, R, 256), lambda r: (0, r, 0)),
          pl.BlockSpec((2, R, 128), lambda r: (0, r, 0)),
          pl.BlockSpec((1, 512), lambda r: (0, 0)),
          pl.BlockSpec((1, 1, R), lambda r: (r, 0, 0)),
          pl.BlockSpec((512, 1024), lambda r: (0, 0)),
          pl.BlockSpec((1, 1024), lambda r: (0, 0)),
          pl.BlockSpec((1024, 128), lambda r: (0, 0)),
          pl.BlockSpec((1, 128), lambda r: (0, 0)),
      ],
      out_specs=pl.BlockSpec((NG, 128), lambda r: (0, 0)),
      out_shape=jax.ShapeDtypeStruct((NG, 128), jnp.float32),
      scratch_shapes=[pltpu.VMEM((NG, 640), jnp.float32)],
  )(acc3, deg2, b3, batch3, wp1, bp1, wp2, bp2)


def kernel(x, edge_index, batch_idx, W1, b1, W2, b2, W3, b3, Wp1, bp1,
           Wp2, bp2):
  loop = jnp.arange(N, dtype=jnp.int32)
  pad = jnp.full((EP - E - N,), NP - 1, dtype=jnp.int32)
  src = jnp.concatenate([edge_index[0], loop, pad])
  dst = jnp.concatenate([edge_index[1], loop, pad])
  dst3 = dst.reshape(NW, CH, W)
  srcF = src.reshape(NSUB, CH2 * W)
  dstF = dst.reshape(NSUB, CH2 * W)
  xp = jnp.pad(x, ((0, NP - N), (0, 0)))
  batch3 = jnp.pad(batch_idx, (0, NP - N),
                   constant_values=NG).reshape(NBLK, 1, R)
  ones128 = jnp.ones((W, 128), jnp.float32)
  zdeg = jnp.zeros((RPS, 128), jnp.float32)
  z128 = jnp.zeros((HR, 128), jnp.float32)
  z256 = jnp.zeros((2 * HR, 128), jnp.float32)

  csrc, cdst, cnt = _make_compact()(srcF, dstF)
  deg2 = _make_deg()(dst3, ones128, zdeg)

  g1 = _dense1(xp, W1, deg2)
  s1 = _make_agg(128, 1, 4)(g1.reshape(NP, 1, 128), csrc, cdst, cnt,
                            z128)

  g2 = _dense2(s1.reshape(NP, 128), deg2, b1.reshape(1, -1), W2)
  s2 = _make_agg(256, 1, 2)(g2.reshape(NP, 2, 128), csrc, cdst, cnt,
                            z256)

  g3 = _dense3(s2.reshape(NP, 256), deg2, b2.reshape(1, -1), W3)
  s3 = _make_agg(256, 2, 2)(g3.reshape(2 * NP, 2, 128), csrc, cdst, cnt,
                            z256)
  s3 = s3.reshape(2, NP, 256)

  return _pool_mlp(s3, deg2, b3.reshape(1, -1), batch3,
                   Wp1, bp1.reshape(1, -1), Wp2, bp2.reshape(1, -1))
